# baseline (device time: 101386 ns/iter reference)
import jax
import jax.numpy as jnp
from jax import lax
from jax.experimental import pallas as pl
from jax.experimental.pallas import tpu as pltpu

N_DEV = 8
B_PER = 512
D = 256
H_BLK = 512
R = 256

BF16 = jnp.bfloat16
F32 = jnp.float32

ORDERS = ((1, 3, 4), (3, 4, 1), (4, 1, 3))
OFF = (0, 192, 352)
LEN = (192, 160, 160)
SIGMA2 = (0, 2, 6, 4, 1, 3, 7, 5)


def kernel(x, Win0, Wout0, Win1, Wout1, Win2, Wout2):
    def body(x_ref, win0_ref, wout0_ref, win1_ref, wout1_ref, win2_ref,
             wout2_ref, out_ref, winf0, winf1, winf2, woutf0, woutf1,
             woutf2, xa, xb,
             gw_ss, gw_rs, go_ss, go_rs, ag_ss_a, ag_rs_a, ag_ss_b, ag_rs_b):
        me = lax.axis_index("i")
        b0 = jnp.bitwise_and(me, 1)
        b1 = jnp.bitwise_and(lax.shift_right_logical(me, 1), 1)
        b2 = jnp.bitwise_and(lax.shift_right_logical(me, 2), 1)
        b01 = jnp.bitwise_xor(b0, b1)
        sl = (me,
              b1 + 2 * b2 + 4 * b01,
              b2 + 2 * b01 + 4 * b1)

        winfs = (winf0, winf1, winf2)
        woutfs = (woutf0, woutf1, woutf2)

        def gather_step(l, s):
            bs = 1 << s
            sends, recvs = [], []
            for p in range(3):
                myb = jnp.bitwise_and(sl[p], N_DEV - bs)
                pb = jnp.bitwise_xor(myb, bs)
                partner = (jnp.bitwise_xor(me, ORDERS[p][s]),)
                sub = pl.ds(OFF[p], LEN[p])
                for buf, ss, rs in ((winfs[l], gw_ss, gw_rs),
                                    (woutfs[l], go_ss, go_rs)):
                    for blk, out in ((myb, sends), (pb, recvs)):
                        out.append(pltpu.make_async_remote_copy(
                            src_ref=buf.at[pl.ds(blk, bs), sub],
                            dst_ref=buf.at[pl.ds(blk, bs), sub],
                            send_sem=ss.at[l, s],
                            recv_sem=rs.at[l, s],
                            device_id=partner,
                            device_id_type=pl.DeviceIdType.MESH,
                        ))
            for d in sends:
                d.start()
            return sends, recvs

        def wait_step(sr):
            sends, recvs = sr
            for d in recvs:
                d.wait_recv()
            for d in sends:
                d.wait_send()

        xloc = x_ref[:].astype(BF16)
        for l, (wi, wo) in enumerate(((win0_ref, wout0_ref),
                                      (win1_ref, wout1_ref),
                                      (win2_ref, wout2_ref))):
            for p in range(3):
                sub = pl.ds(OFF[p], LEN[p])
                winfs[l][sl[p], sub] = wi[sub]
                woutfs[l][sl[p], sub] = wo[sub].astype(BF16)

        def hdot(xv, wslice):
            return lax.dot_general(
                xv, wslice, (((1,), (1,)), ((), ())),
                preferred_element_type=F32)

        def layer_block(l, c, xv, acc):
            h = jnp.maximum(hdot(xv, winfs[l][c]), 0.0).astype(BF16)
            contrib = jnp.dot(h, woutfs[l][c], preferred_element_type=F32)
            return contrib if acc is None else acc + contrib

        for s in range(3):
            wait_step(gather_step(0, s))

        sr = gather_step(1, 0)
        acc = None
        for c in range(3):
            acc = layer_block(0, c, xloc, acc)
        wait_step(sr)
        sr = gather_step(1, 1)
        for c in range(3, 6):
            acc = layer_block(0, c, xloc, acc)
        wait_step(sr)
        sr = gather_step(1, 2)
        for c in range(6, 8):
            acc = layer_block(0, c, xloc, acc)
        wait_step(sr)

        x1 = acc.astype(BF16)
        sr = gather_step(2, 0)
        acc = None
        for c in range(3):
            acc = layer_block(1, c, x1, acc)
        wait_step(sr)
        sr = gather_step(2, 1)
        for c in range(3, 6):
            acc = layer_block(1, c, x1, acc)
        wait_step(sr)
        sr = gather_step(2, 2)
        for c in range(6, 8):
            acc = layer_block(1, c, x1, acc)
        wait_step(sr)

        x2 = acc.astype(BF16)

        def ag_step(s, xbuf, own, masks, ss, rs):
            bs = 1 << s
            myb = jnp.bitwise_and(own, N_DEV - bs)
            pb = jnp.bitwise_xor(myb, bs)
            partner = (jnp.bitwise_xor(me, masks[s]),)
            send_d = pltpu.make_async_remote_copy(
                src_ref=xbuf.at[pl.ds(myb * R, bs * R)],
                dst_ref=xbuf.at[pl.ds(myb * R, bs * R)],
                send_sem=ss.at[s], recv_sem=rs.at[s],
                device_id=partner, device_id_type=pl.DeviceIdType.MESH,
            )
            send_d.start()
            recv_d = pltpu.make_async_remote_copy(
                src_ref=xbuf.at[pl.ds(pb * R, bs * R)],
                dst_ref=xbuf.at[pl.ds(pb * R, bs * R)],
                send_sem=ss.at[s], recv_sem=rs.at[s],
                device_id=partner, device_id_type=pl.DeviceIdType.MESH,
            )
            return send_d, recv_d

        accA = None
        for c in range(N_DEV):
            accA = layer_block(2, c, x2[:R], accA)
        xa[pl.ds(me * R, R)] = accA.astype(BF16)
        a0 = ag_step(0, xa, me, ORDERS[0], ag_ss_a, ag_rs_a)

        accB = None
        for c in range(N_DEV):
            accB = layer_block(2, c, x2[R:], accB)
        xb[pl.ds(sl[2] * R, R)] = accB.astype(BF16)

        a0[1].wait_recv()
        a0[0].wait_send()
        a1 = ag_step(1, xa, me, ORDERS[0], ag_ss_a, ag_rs_a)
        b0 = ag_step(0, xb, sl[2], ORDERS[2], ag_ss_b, ag_rs_b)
        a1[1].wait_recv()
        b0[1].wait_recv()
        a1[0].wait_send()
        b0[0].wait_send()
        a2 = ag_step(2, xa, me, ORDERS[0], ag_ss_a, ag_rs_a)
        b1 = ag_step(1, xb, sl[2], ORDERS[2], ag_ss_b, ag_rs_b)
        a2[1].wait_recv()
        b1[1].wait_recv()
        a2[0].wait_send()
        b1[0].wait_send()
        b2 = ag_step(2, xb, sl[2], ORDERS[2], ag_ss_b, ag_rs_b)
        b2[1].wait_recv()
        b2[0].wait_send()

        for c in range(N_DEV):
            out_ref[pl.ds(c * 2 * R, R), :] = xa[pl.ds(c * R, R)].astype(F32)
            out_ref[pl.ds(c * 2 * R + R, R), :] = (
                xb[pl.ds(SIGMA2[c] * R, R)].astype(F32))

    sem3 = pltpu.SemaphoreType.DMA((3,))
    sem33 = pltpu.SemaphoreType.DMA((3, 3))
    wshape = pltpu.VMEM((N_DEV, H_BLK, D), BF16)
    return pl.pallas_call(
        body,
        out_shape=jax.ShapeDtypeStruct((N_DEV * B_PER, D), F32),
        in_specs=[pl.BlockSpec(memory_space=pltpu.VMEM)] * 7,
        out_specs=pl.BlockSpec(memory_space=pltpu.VMEM),
        scratch_shapes=[
            wshape, wshape, wshape,
            wshape, wshape, wshape,
            pltpu.VMEM((N_DEV * R, D), BF16),
            pltpu.VMEM((N_DEV * R, D), BF16),
            sem33, sem33,
            sem33, sem33,
            sem3, sem3, sem3, sem3,
        ],
    )(x, Win0.T.astype(BF16), Wout0, Win1.T.astype(BF16), Wout1,
      Win2.T.astype(BF16), Wout2)


# device time: 87154 ns/iter; 1.1633x vs baseline; 1.1633x over previous
import jax
import jax.numpy as jnp
from jax import lax
from jax.experimental import pallas as pl
from jax.experimental.pallas import tpu as pltpu

N_DEV = 8
B_PER = 512
D = 256
H_BLK = 512
R = 256

BF16 = jnp.bfloat16
F32 = jnp.float32

ORDERS = ((1, 3, 4), (3, 4, 1), (4, 1, 3))
SIGMA2 = (0, 2, 6, 4, 1, 3, 7, 5)


def kernel(x, Win0, Wout0, Win1, Wout1, Win2, Wout2):
    def body(x_ref, win0_ref, wout0_ref, win1_ref, wout1_ref, win2_ref,
             wout2_ref, out_ref, winf0, winf1, winf2, woutf0, woutf1,
             woutf2, xa, xb,
             gw_ss, gw_rs, go_ss, go_rs, ag_ss_a, ag_rs_a, ag_ss_b, ag_rs_b):
        me = lax.axis_index("i")
        b0 = jnp.bitwise_and(me, 1)
        b1 = jnp.bitwise_and(lax.shift_right_logical(me, 1), 1)
        b2 = jnp.bitwise_and(lax.shift_right_logical(me, 2), 1)
        b01 = jnp.bitwise_xor(b0, b1)
        sl = (me,
              b1 + 2 * b2 + 4 * b01,
              b2 + 2 * b01 + 4 * b1)

        winfs = (winf0, winf1, winf2)
        woutfs = (woutf0, woutf1, woutf2)

        def gather_step(l, s):
            bs = 1 << s
            myb = jnp.bitwise_and(sl[l], N_DEV - bs)
            pb = jnp.bitwise_xor(myb, bs)
            partner = (jnp.bitwise_xor(me, ORDERS[l][s]),)

            def descs(blk):
                cols = pl.ds(blk * H_BLK, bs * H_BLK)
                w_d = pltpu.make_async_remote_copy(
                    src_ref=winfs[l].at[cols],
                    dst_ref=winfs[l].at[cols],
                    send_sem=gw_ss.at[l, s],
                    recv_sem=gw_rs.at[l, s],
                    device_id=partner,
                    device_id_type=pl.DeviceIdType.MESH,
                )
                o_d = pltpu.make_async_remote_copy(
                    src_ref=woutfs[l].at[cols],
                    dst_ref=woutfs[l].at[cols],
                    send_sem=go_ss.at[l, s],
                    recv_sem=go_rs.at[l, s],
                    device_id=partner,
                    device_id_type=pl.DeviceIdType.MESH,
                )
                return w_d, o_d

            ws, os_ = descs(myb)
            ws.start()
            os_.start()
            wr, orr = descs(pb)
            return ws, os_, wr, orr

        def wait_step(ds4):
            ws, os_, wr, orr = ds4
            wr.wait_recv()
            orr.wait_recv()
            ws.wait_send()
            os_.wait_send()

        xloc = x_ref[:].astype(BF16)
        for l, (wi, wo) in enumerate(((win0_ref, wout0_ref),
                                      (win1_ref, wout1_ref),
                                      (win2_ref, wout2_ref))):
            cols = pl.ds(sl[l] * H_BLK, H_BLK)
            winfs[l][cols] = wi[:]
            woutfs[l][cols] = wo[:].astype(BF16)

        def hdot(xv, wslice):
            return lax.dot_general(
                xv, wslice, (((1,), (1,)), ((), ())),
                preferred_element_type=F32)

        def l0_block(acc, blk, nb):
            cols = pl.ds(blk * H_BLK, nb * H_BLK)
            h = hdot(xloc, winf0[cols])
            h = jnp.maximum(h, 0.0).astype(BF16)
            c = jnp.dot(h, woutf0[cols], preferred_element_type=F32)
            return c if acc is None else acc + c

        st0 = [gather_step(l, 0) for l in range(3)]
        acc = l0_block(None, sl[0], 1)
        for d4 in st0:
            wait_step(d4)
        st1 = [gather_step(l, 1) for l in range(3)]
        acc = l0_block(acc, jnp.bitwise_xor(sl[0], 1), 1)
        for d4 in st1:
            wait_step(d4)
        st2 = [gather_step(l, 2) for l in range(3)]
        blk1 = jnp.bitwise_xor(jnp.bitwise_and(sl[0], N_DEV - 2), 2)
        acc = l0_block(acc, blk1, 2)
        for d4 in st2:
            wait_step(d4)
        blk2 = jnp.bitwise_xor(jnp.bitwise_and(sl[0], N_DEV - 4), 4)
        acc = l0_block(acc, blk2, 4)

        x1b = acc.astype(BF16)
        h = jnp.maximum(hdot(x1b, winfs[1][:]), 0.0).astype(BF16)
        x2b = jnp.dot(h, woutfs[1][:], preferred_element_type=F32).astype(BF16)

        def ag_step(s, xbuf, own, masks, ss, rs):
            bs = 1 << s
            myb = jnp.bitwise_and(own, N_DEV - bs)
            pb = jnp.bitwise_xor(myb, bs)
            partner = (jnp.bitwise_xor(me, masks[s]),)
            send_d = pltpu.make_async_remote_copy(
                src_ref=xbuf.at[pl.ds(myb * R, bs * R)],
                dst_ref=xbuf.at[pl.ds(myb * R, bs * R)],
                send_sem=ss.at[s],
                recv_sem=rs.at[s],
                device_id=partner,
                device_id_type=pl.DeviceIdType.MESH,
            )
            send_d.start()
            recv_d = pltpu.make_async_remote_copy(
                src_ref=xbuf.at[pl.ds(pb * R, bs * R)],
                dst_ref=xbuf.at[pl.ds(pb * R, bs * R)],
                send_sem=ss.at[s],
                recv_sem=rs.at[s],
                device_id=partner,
                device_id_type=pl.DeviceIdType.MESH,
            )
            return send_d, recv_d

        sbf = sl[2]
        hA = jnp.maximum(hdot(x2b[:R], winfs[2][:]), 0.0).astype(BF16)
        accA = jnp.dot(hA, woutfs[2][:], preferred_element_type=F32)
        xa[pl.ds(me * R, R)] = accA.astype(BF16)
        a0 = ag_step(0, xa, me, ORDERS[0], ag_ss_a, ag_rs_a)

        hB = jnp.maximum(hdot(x2b[R:], winfs[2][:]), 0.0).astype(BF16)
        accB = jnp.dot(hB, woutfs[2][:], preferred_element_type=F32)
        xb[pl.ds(sbf * R, R)] = accB.astype(BF16)

        a0[1].wait_recv()
        a0[0].wait_send()
        a1 = ag_step(1, xa, me, ORDERS[0], ag_ss_a, ag_rs_a)
        b0 = ag_step(0, xb, sbf, ORDERS[2], ag_ss_b, ag_rs_b)
        a1[1].wait_recv()
        b0[1].wait_recv()
        a1[0].wait_send()
        b0[0].wait_send()
        a2 = ag_step(2, xa, me, ORDERS[0], ag_ss_a, ag_rs_a)
        b1 = ag_step(1, xb, sbf, ORDERS[2], ag_ss_b, ag_rs_b)
        a2[1].wait_recv()
        b1[1].wait_recv()
        a2[0].wait_send()
        b1[0].wait_send()
        b2 = ag_step(2, xb, sbf, ORDERS[2], ag_ss_b, ag_rs_b)
        b2[1].wait_recv()
        b2[0].wait_send()

        for c in range(N_DEV):
            out_ref[pl.ds(c * 2 * R, R), :] = xa[pl.ds(c * R, R)]
            out_ref[pl.ds(c * 2 * R + R, R), :] = xb[pl.ds(SIGMA2[c] * R, R)]

    sem3 = pltpu.SemaphoreType.DMA((3,))
    sem33 = pltpu.SemaphoreType.DMA((3, 3))
    return pl.pallas_call(
        body,
        out_shape=jax.ShapeDtypeStruct((N_DEV * B_PER, D), BF16),
        in_specs=[pl.BlockSpec(memory_space=pltpu.VMEM)] * 7,
        out_specs=pl.BlockSpec(memory_space=pltpu.VMEM),
        scratch_shapes=[
            pltpu.VMEM((N_DEV * H_BLK, D), BF16),
            pltpu.VMEM((N_DEV * H_BLK, D), BF16),
            pltpu.VMEM((N_DEV * H_BLK, D), BF16),
            pltpu.VMEM((N_DEV * H_BLK, D), BF16),
            pltpu.VMEM((N_DEV * H_BLK, D), BF16),
            pltpu.VMEM((N_DEV * H_BLK, D), BF16),
            pltpu.VMEM((N_DEV * R, D), BF16),
            pltpu.VMEM((N_DEV * R, D), BF16),
            sem33, sem33,
            sem33, sem33,
            sem3, sem3, sem3, sem3,
        ],
    )(x, Win0.T.astype(BF16), Wout0, Win1.T.astype(BF16), Wout1,
      Win2.T.astype(BF16), Wout2)
